# baseline (speedup 1.0000x reference)
import functools

import jax
import jax.numpy as jnp
from jax import lax
from jax.experimental import pallas as pl
from jax.experimental.pallas import tpu as pltpu
from jax.experimental.pallas import tpu_sc as plsc

N_NODES = 10000
N_EDGES = 320000
NFEAT = 128
NHID = 128
NCLASS = 64

E_PAD = 327680
C = 128
ROWS_PER_TILE = N_NODES // 16
_ZCHUNKS = [(0, 128), (128, 128), (256, 128), (384, 128), (512, 113)]


def _mm_body(x_ref, w_ref, o_ref):
    o_ref[...] = jnp.dot(x_ref[...], w_ref[...],
                         preferred_element_type=jnp.float32)


def _matmul(x, w, bm=1000):
    m, k = x.shape
    n = w.shape[1]
    return pl.pallas_call(
        _mm_body,
        grid=(m // bm,),
        in_specs=[pl.BlockSpec((bm, k), lambda i: (i, 0)),
                  pl.BlockSpec((k, n), lambda i: (0, 0))],
        out_specs=pl.BlockSpec((bm, n), lambda i: (i, 0)),
        out_shape=jax.ShapeDtypeStruct((m, n), jnp.float32),
    )(x, w)


def _mm2_body(p_ref, w_ref, o_ref):
    h = jnp.maximum(p_ref[0] + p_ref[1], 0.0)
    o_ref[...] = jnp.dot(h, w_ref[...], preferred_element_type=jnp.float32)


def _relu_sum_matmul(p, w, bm=1000):
    _, m, k = p.shape
    n = w.shape[1]
    return pl.pallas_call(
        _mm2_body,
        grid=(m // bm,),
        in_specs=[pl.BlockSpec((2, bm, k), lambda i: (0, i, 0)),
                  pl.BlockSpec((k, n), lambda i: (0, 0))],
        out_specs=pl.BlockSpec((bm, n), lambda i: (i, 0)),
        out_shape=jax.ShapeDtypeStruct((m, n), jnp.float32),
    )(p, w)


def _sum2_body(p_ref, o_ref):
    o_ref[...] = p_ref[0] + p_ref[1]


def _sum2(p, bm=1000):
    _, m, n = p.shape
    return pl.pallas_call(
        _sum2_body,
        grid=(m // bm,),
        in_specs=[pl.BlockSpec((2, bm, n), lambda i: (0, i, 0))],
        out_specs=pl.BlockSpec((bm, n), lambda i: (i, 0)),
        out_shape=jax.ShapeDtypeStruct((m, n), jnp.float32),
    )(p)


_GATHER_DNUMS = lax.GatherDimensionNumbers(
    offset_dims=(), collapsed_slice_dims=(0,), start_index_map=(0,))


def _lane_splat(vec, lane):
    idx = jnp.full((16, 1), lane, jnp.int32)
    return lax.gather(vec, idx, _GATHER_DNUMS, slice_sizes=(1,),
                      mode=lax.GatherScatterMode.PROMISE_IN_BOUNDS)


def _make_agg(F):
    nslab = 32
    ept = E_PAD // nslab
    nch = ept // C
    mesh = plsc.VectorSubcoreMesh(core_axis_name="c", subcore_axis_name="s")

    @functools.partial(
        pl.kernel,
        out_type=jax.ShapeDtypeStruct((2, N_NODES, F), jnp.float32),
        mesh=mesh,
        compiler_params=pltpu.CompilerParams(use_tc_tiling_on_sc=False,
                                             needs_layout_passes=False),
        scratch_types=(
            [pltpu.VMEM_SHARED((N_NODES, F), jnp.float32),
             pltpu.VMEM_SHARED((N_NODES, F), jnp.float32),
             pltpu.VMEM((4, 3, C), jnp.int32),
             pltpu.VMEM((4, C), jnp.int32)]
            + [pltpu.VMEM((C, F), jnp.float32) for _ in range(4)]
            + [pltpu.SemaphoreType.DMA] * 12
        ),
    )
    def agg(h_hbm, edges_hbm, out_hbm,
            acc_sh, h_sh, stage_v, dstc_v, rows0_v, rows1_v, rows2_v, rows3_v,
            *sems):
        c = lax.axis_index("c")
        s = lax.axis_index("s")
        wid = s * 2 + c
        rows = (rows0_v, rows1_v, rows2_v, rows3_v)
        ssems, gsems, scsems = sems[0:4], sems[4:8], sems[8:12]
        h_view = h_sh

        def stage_start(k, b):
            pltpu.async_copy(edges_hbm.at[wid, k], stage_v.at[b], ssems[b])

        def stage_wait(k, b):
            pltpu.make_async_copy(edges_hbm.at[wid, k], stage_v.at[b],
                                  ssems[b]).wait()

        def gather_start(b):
            pltpu.async_copy(h_view.at[stage_v.at[b, 0]], rows[b], gsems[b])

        def gather_wait(b):
            pltpu.make_async_copy(h_view.at[stage_v.at[b, 0]], rows[b],
                                  gsems[b]).wait()

        def scatter_start(b):
            pltpu.async_copy(rows[b], acc_sh.at[dstc_v.at[b]], scsems[b],
                             add=True)

        def scatter_wait(b):
            pltpu.make_async_copy(rows[b], acc_sh.at[dstc_v.at[b]],
                                  scsems[b]).wait()

        for b in range(4):
            stage_start(b, b)

        def zrow(r, carry):
            for j in range(F // 16):
                rows0_v[r, pl.ds(j * 16, 16)] = jnp.zeros((16,), jnp.float32)
            return carry
        lax.fori_loop(0, C, zrow, 0)
        for (z0, zn) in _ZCHUNKS:
            r0 = s * ROWS_PER_TILE + z0
            pltpu.sync_copy(rows0_v.at[pl.ds(0, zn)],
                            acc_sh.at[pl.ds(r0, zn)])
            pltpu.sync_copy(h_hbm.at[pl.ds(r0, zn)], h_sh.at[pl.ds(r0, zn)])
        plsc.subcore_barrier()

        stage_wait(0, 0)
        gather_start(0)
        stage_wait(1, 1)
        gather_start(1)

        def step(kk, carry):
            for b in range(4):
                k = kk * 4 + b
                gather_wait(b)

                def group(g, carry2):
                    wv = plsc.bitcast(stage_v[b, 2, pl.ds(g * 16, 16)],
                                      jnp.float32)
                    for l in range(16):
                        splat = _lane_splat(wv, l)
                        e = g * 16 + l
                        for j in range(F // 16):
                            rows[b][e, pl.ds(j * 16, 16)] = (
                                rows[b][e, pl.ds(j * 16, 16)] * splat)
                    return carry2
                lax.fori_loop(0, C // 16, group, 0)

                for j in range(C // 16):
                    dstc_v[b, pl.ds(j * 16, 16)] = stage_v[b, 1,
                                                           pl.ds(j * 16, 16)]
                scatter_start(b)

                @pl.when(k + 4 < nch)
                def _():
                    stage_start(k + 4, b)

                if b < 2:
                    @pl.when(kk >= 1)
                    def _():
                        scatter_wait((b + 2) % 4)
                else:
                    scatter_wait((b + 2) % 4)

                @pl.when(k + 2 < nch)
                def _():
                    stage_wait(k + 2, (b + 2) % 4)
                    gather_start((b + 2) % 4)
            return carry
        lax.fori_loop(0, nch // 4, step, 0)
        scatter_wait(2)
        scatter_wait(3)

        plsc.subcore_barrier()
        for (z0, zn) in _ZCHUNKS:
            r0 = s * ROWS_PER_TILE + z0
            pltpu.sync_copy(acc_sh.at[pl.ds(r0, zn)],
                            out_hbm.at[c, pl.ds(r0, zn)])

    return agg


def _make_agg_2buf(F):
    nslab = 32
    ept = E_PAD // nslab
    nch = ept // C
    mesh = plsc.VectorSubcoreMesh(core_axis_name="c", subcore_axis_name="s")

    @functools.partial(
        pl.kernel,
        out_type=jax.ShapeDtypeStruct((2, N_NODES, F), jnp.float32),
        mesh=mesh,
        compiler_params=pltpu.CompilerParams(use_tc_tiling_on_sc=False,
                                             needs_layout_passes=False),
        scratch_types=(
            [pltpu.VMEM_SHARED((N_NODES, F), jnp.float32),
             pltpu.VMEM((4, 3, C), jnp.int32)]
            + [pltpu.VMEM((C, F), jnp.float32) for _ in range(2)]
            + [pltpu.SemaphoreType.DMA] * 6
        ),
    )
    def agg(h_hbm, edges_hbm, out_hbm,
            acc_sh, stage_v, rows0_v, rows1_v, *sems):
        c = lax.axis_index("c")
        s = lax.axis_index("s")
        wid = s * 2 + c
        rows = (rows0_v, rows1_v)
        ssems, gsems = sems[0:4], sems[4:6]

        def stage_start(k, b):
            pltpu.async_copy(edges_hbm.at[wid, k], stage_v.at[b], ssems[b])

        def stage_wait(k, b):
            pltpu.make_async_copy(edges_hbm.at[wid, k], stage_v.at[b],
                                  ssems[b]).wait()

        def gather_start(sb, rb):
            pltpu.async_copy(h_hbm.at[stage_v.at[sb, 0]], rows[rb], gsems[rb])

        def gather_wait(sb, rb):
            pltpu.make_async_copy(h_hbm.at[stage_v.at[sb, 0]], rows[rb],
                                  gsems[rb]).wait()

        for sb in range(4):
            stage_start(sb, sb)

        def zrow(r, carry):
            for j in range(F // 16):
                rows0_v[r, pl.ds(j * 16, 16)] = jnp.zeros((16,), jnp.float32)
            return carry
        lax.fori_loop(0, C, zrow, 0)
        for (z0, zn) in _ZCHUNKS:
            pltpu.sync_copy(rows0_v.at[pl.ds(0, zn)],
                            acc_sh.at[pl.ds(s * ROWS_PER_TILE + z0, zn)])
        plsc.subcore_barrier()

        stage_wait(0, 0)
        gather_start(0, 0)
        stage_wait(1, 1)
        gather_start(1, 1)

        def step(kk, carry):
            for b in range(4):
                k = kk * 4 + b
                rb = b % 2
                sb2 = (b + 2) % 4
                gather_wait(b, rb)

                def group(g, carry2):
                    wv = plsc.bitcast(stage_v[b, 2, pl.ds(g * 16, 16)],
                                      jnp.float32)
                    for l in range(16):
                        splat = _lane_splat(wv, l)
                        e = g * 16 + l
                        for j in range(F // 16):
                            rows[rb][e, pl.ds(j * 16, 16)] = (
                                rows[rb][e, pl.ds(j * 16, 16)] * splat)
                    return carry2
                lax.fori_loop(0, C // 16, group, 0)

                pltpu.sync_copy(rows[rb], acc_sh.at[stage_v.at[b, 1]],
                                add=True)

                @pl.when(k + 4 < nch)
                def _():
                    stage_start(k + 4, b)

                @pl.when(k + 2 < nch)
                def _():
                    stage_wait(k + 2, sb2)
                    gather_start(sb2, rb)
            return carry
        lax.fori_loop(0, nch // 4, step, 0)

        plsc.subcore_barrier()
        for (z0, zn) in _ZCHUNKS:
            r0 = s * ROWS_PER_TILE + z0
            pltpu.sync_copy(acc_sh.at[pl.ds(r0, zn)],
                            out_hbm.at[c, pl.ds(r0, zn)])

    return agg


_agg_l1 = _make_agg_2buf(NHID)
_agg_l2 = _make_agg(NCLASS)


def _pad_edges(ei, ew, nslab):
    npad = E_PAD - N_EDGES
    nch = E_PAD // nslab // C
    spread = jnp.arange(npad, dtype=jnp.int32) % N_NODES
    src = jnp.concatenate([ei[0], spread]).reshape(nslab, nch, 1, C)
    dst = jnp.concatenate([ei[1], spread]).reshape(nslab, nch, 1, C)
    w = lax.bitcast_convert_type(
        jnp.pad(ew, (0, npad)), jnp.int32).reshape(nslab, nch, 1, C)
    return jnp.concatenate([src, dst, w], axis=2)


def kernel(x, edge_index1, edge_index2, edge_weight1, edge_weight2, W1, W2):
    e1 = _pad_edges(edge_index1, edge_weight1, 32)
    e2 = _pad_edges(edge_index2, edge_weight2, 32)

    h1 = _matmul(x, W1)
    p1 = _agg_l1(h1, e1)
    h2 = _relu_sum_matmul(p1, W2)
    p2 = _agg_l2(h2, e2)
    return _sum2(p2)


# speedup vs baseline: 1.0040x; 1.0040x over previous
import functools

import jax
import jax.numpy as jnp
from jax import lax
from jax.experimental import pallas as pl
from jax.experimental.pallas import tpu as pltpu
from jax.experimental.pallas import tpu_sc as plsc

N_NODES = 10000
N_EDGES = 320000
NFEAT = 128
NHID = 128
NCLASS = 64

E_PAD = 327680
C = 128
ROWS_PER_TILE = N_NODES // 16
_ZCHUNKS = [(0, 128), (128, 128), (256, 128), (384, 128), (512, 113)]


def _mm_body(x_ref, w_ref, o_ref):
    o_ref[...] = jnp.dot(x_ref[...], w_ref[...],
                         preferred_element_type=jnp.float32)


def _matmul(x, w, bm=1000):
    m, k = x.shape
    n = w.shape[1]
    return pl.pallas_call(
        _mm_body,
        grid=(m // bm,),
        in_specs=[pl.BlockSpec((bm, k), lambda i: (i, 0)),
                  pl.BlockSpec((k, n), lambda i: (0, 0))],
        out_specs=pl.BlockSpec((bm, n), lambda i: (i, 0)),
        out_shape=jax.ShapeDtypeStruct((m, n), jnp.float32),
    )(x, w)


def _mm2_body(p_ref, w_ref, o_ref):
    h = jnp.maximum(p_ref[0] + p_ref[1], 0.0)
    o_ref[...] = jnp.dot(h, w_ref[...], preferred_element_type=jnp.float32)


def _relu_sum_matmul(p, w, bm=1000):
    _, m, k = p.shape
    n = w.shape[1]
    return pl.pallas_call(
        _mm2_body,
        grid=(m // bm,),
        in_specs=[pl.BlockSpec((2, bm, k), lambda i: (0, i, 0)),
                  pl.BlockSpec((k, n), lambda i: (0, 0))],
        out_specs=pl.BlockSpec((bm, n), lambda i: (i, 0)),
        out_shape=jax.ShapeDtypeStruct((m, n), jnp.float32),
    )(p, w)


def _sum2_body(p_ref, o_ref):
    o_ref[...] = p_ref[0] + p_ref[1]


def _sum2(p, bm=1000):
    _, m, n = p.shape
    return pl.pallas_call(
        _sum2_body,
        grid=(m // bm,),
        in_specs=[pl.BlockSpec((2, bm, n), lambda i: (0, i, 0))],
        out_specs=pl.BlockSpec((bm, n), lambda i: (i, 0)),
        out_shape=jax.ShapeDtypeStruct((m, n), jnp.float32),
    )(p)


_GATHER_DNUMS = lax.GatherDimensionNumbers(
    offset_dims=(), collapsed_slice_dims=(0,), start_index_map=(0,))


def _lane_splat(vec, lane):
    idx = jnp.full((16, 1), lane, jnp.int32)
    return lax.gather(vec, idx, _GATHER_DNUMS, slice_sizes=(1,),
                      mode=lax.GatherScatterMode.PROMISE_IN_BOUNDS)


def _make_agg(F):
    nslab = 32
    ept = E_PAD // nslab
    nch = ept // C
    R = 8
    LA = 6
    mesh = plsc.VectorSubcoreMesh(core_axis_name="c", subcore_axis_name="s")

    @functools.partial(
        pl.kernel,
        out_type=jax.ShapeDtypeStruct((2, N_NODES, F), jnp.float32),
        mesh=mesh,
        compiler_params=pltpu.CompilerParams(use_tc_tiling_on_sc=False,
                                             needs_layout_passes=False),
        scratch_types=(
            [pltpu.VMEM_SHARED((N_NODES, F), jnp.float32),
             pltpu.VMEM((R, 3, C), jnp.int32),
             pltpu.VMEM((R, C), jnp.int32)]
            + [pltpu.VMEM((C, F), jnp.float32) for _ in range(R)]
            + [pltpu.SemaphoreType.DMA] * (3 * R)
        ),
    )
    def agg(h_hbm, edges_hbm, out_hbm, acc_sh, stage_v, dstc_v, *rest):
        rows, sems = rest[:R], rest[R:]
        ssems, gsems, scsems = sems[0:R], sems[R:2 * R], sems[2 * R:3 * R]
        c = lax.axis_index("c")
        s = lax.axis_index("s")
        wid = s * 2 + c

        def stage_start(k, b):
            pltpu.async_copy(edges_hbm.at[wid, k], stage_v.at[b], ssems[b])

        def stage_wait(k, b):
            pltpu.make_async_copy(edges_hbm.at[wid, k], stage_v.at[b],
                                  ssems[b]).wait()

        def gather_start(b):
            pltpu.async_copy(h_hbm.at[stage_v.at[b, 0]], rows[b], gsems[b])

        def gather_wait(b):
            pltpu.make_async_copy(h_hbm.at[stage_v.at[b, 0]], rows[b],
                                  gsems[b]).wait()

        def scatter_start(b):
            pltpu.async_copy(rows[b], acc_sh.at[dstc_v.at[b]], scsems[b],
                             add=True)

        def scatter_wait(b):
            pltpu.make_async_copy(rows[b], acc_sh.at[dstc_v.at[b]],
                                  scsems[b]).wait()

        for b in range(R):
            stage_start(b, b)

        def zrow(r, carry):
            for j in range(F // 16):
                rows[0][r, pl.ds(j * 16, 16)] = jnp.zeros((16,), jnp.float32)
            return carry
        lax.fori_loop(0, C, zrow, 0)
        for (z0, zn) in _ZCHUNKS:
            pltpu.sync_copy(rows[0].at[pl.ds(0, zn)],
                            acc_sh.at[pl.ds(s * ROWS_PER_TILE + z0, zn)])
        plsc.subcore_barrier()

        for b in range(LA):
            stage_wait(b, b)
            gather_start(b)

        def step(kk, carry):
            for b in range(R):
                k = kk * R + b
                gather_wait(b)

                def group(g, carry2):
                    wv = plsc.bitcast(stage_v[b, 2, pl.ds(g * 16, 16)],
                                      jnp.float32)
                    for l in range(16):
                        splat = _lane_splat(wv, l)
                        e = g * 16 + l
                        for j in range(F // 16):
                            rows[b][e, pl.ds(j * 16, 16)] = (
                                rows[b][e, pl.ds(j * 16, 16)] * splat)
                    return carry2
                lax.fori_loop(0, C // 16, group, 0)

                for j in range(C // 16):
                    dstc_v[b, pl.ds(j * 16, 16)] = stage_v[b, 1,
                                                           pl.ds(j * 16, 16)]
                scatter_start(b)

                @pl.when(k + R < nch)
                def _():
                    stage_start(k + R, b)

                if b < 2:
                    @pl.when(kk >= 1)
                    def _():
                        scatter_wait((b - 2) % R)
                else:
                    scatter_wait((b - 2) % R)

                @pl.when(k + LA < nch)
                def _():
                    stage_wait(k + LA, (b + LA) % R)
                    gather_start((b + LA) % R)
            return carry
        lax.fori_loop(0, nch // R, step, 0)
        scatter_wait((nch - 2) % R)
        scatter_wait((nch - 1) % R)

        plsc.subcore_barrier()
        for (z0, zn) in _ZCHUNKS:
            r0 = s * ROWS_PER_TILE + z0
            pltpu.sync_copy(acc_sh.at[pl.ds(r0, zn)],
                            out_hbm.at[c, pl.ds(r0, zn)])

    return agg


def _make_agg_2buf(F):
    nslab = 32
    ept = E_PAD // nslab
    nch = ept // C
    mesh = plsc.VectorSubcoreMesh(core_axis_name="c", subcore_axis_name="s")

    @functools.partial(
        pl.kernel,
        out_type=jax.ShapeDtypeStruct((2, N_NODES, F), jnp.float32),
        mesh=mesh,
        compiler_params=pltpu.CompilerParams(use_tc_tiling_on_sc=False,
                                             needs_layout_passes=False),
        scratch_types=(
            [pltpu.VMEM_SHARED((N_NODES, F), jnp.float32),
             pltpu.VMEM((4, 3, C), jnp.int32)]
            + [pltpu.VMEM((C, F), jnp.float32) for _ in range(2)]
            + [pltpu.SemaphoreType.DMA] * 6
        ),
    )
    def agg(h_hbm, edges_hbm, out_hbm,
            acc_sh, stage_v, rows0_v, rows1_v, *sems):
        c = lax.axis_index("c")
        s = lax.axis_index("s")
        wid = s * 2 + c
        rows = (rows0_v, rows1_v)
        ssems, gsems = sems[0:4], sems[4:6]

        def stage_start(k, b):
            pltpu.async_copy(edges_hbm.at[wid, k], stage_v.at[b], ssems[b])

        def stage_wait(k, b):
            pltpu.make_async_copy(edges_hbm.at[wid, k], stage_v.at[b],
                                  ssems[b]).wait()

        def gather_start(sb, rb):
            pltpu.async_copy(h_hbm.at[stage_v.at[sb, 0]], rows[rb], gsems[rb])

        def gather_wait(sb, rb):
            pltpu.make_async_copy(h_hbm.at[stage_v.at[sb, 0]], rows[rb],
                                  gsems[rb]).wait()

        for sb in range(4):
            stage_start(sb, sb)

        def zrow(r, carry):
            for j in range(F // 16):
                rows0_v[r, pl.ds(j * 16, 16)] = jnp.zeros((16,), jnp.float32)
            return carry
        lax.fori_loop(0, C, zrow, 0)
        for (z0, zn) in _ZCHUNKS:
            pltpu.sync_copy(rows0_v.at[pl.ds(0, zn)],
                            acc_sh.at[pl.ds(s * ROWS_PER_TILE + z0, zn)])
        plsc.subcore_barrier()

        stage_wait(0, 0)
        gather_start(0, 0)
        stage_wait(1, 1)
        gather_start(1, 1)

        def step(kk, carry):
            for b in range(4):
                k = kk * 4 + b
                rb = b % 2
                sb2 = (b + 2) % 4
                gather_wait(b, rb)

                def group(g, carry2):
                    wv = plsc.bitcast(stage_v[b, 2, pl.ds(g * 16, 16)],
                                      jnp.float32)
                    for l in range(16):
                        splat = _lane_splat(wv, l)
                        e = g * 16 + l
                        for j in range(F // 16):
                            rows[rb][e, pl.ds(j * 16, 16)] = (
                                rows[rb][e, pl.ds(j * 16, 16)] * splat)
                    return carry2
                lax.fori_loop(0, C // 16, group, 0)

                pltpu.sync_copy(rows[rb], acc_sh.at[stage_v.at[b, 1]],
                                add=True)

                @pl.when(k + 4 < nch)
                def _():
                    stage_start(k + 4, b)

                @pl.when(k + 2 < nch)
                def _():
                    stage_wait(k + 2, sb2)
                    gather_start(sb2, rb)
            return carry
        lax.fori_loop(0, nch // 4, step, 0)

        plsc.subcore_barrier()
        for (z0, zn) in _ZCHUNKS:
            r0 = s * ROWS_PER_TILE + z0
            pltpu.sync_copy(acc_sh.at[pl.ds(r0, zn)],
                            out_hbm.at[c, pl.ds(r0, zn)])

    return agg


_agg_l1 = _make_agg_2buf(NHID)
_agg_l2 = _make_agg(NCLASS)


def _pad_edges(ei, ew, nslab):
    npad = E_PAD - N_EDGES
    nch = E_PAD // nslab // C
    spread = jnp.arange(npad, dtype=jnp.int32) % N_NODES
    src = jnp.concatenate([ei[0], spread]).reshape(nslab, nch, 1, C)
    dst = jnp.concatenate([ei[1], spread]).reshape(nslab, nch, 1, C)
    w = lax.bitcast_convert_type(
        jnp.pad(ew, (0, npad)), jnp.int32).reshape(nslab, nch, 1, C)
    return jnp.concatenate([src, dst, w], axis=2)


def kernel(x, edge_index1, edge_index2, edge_weight1, edge_weight2, W1, W2):
    e1 = _pad_edges(edge_index1, edge_weight1, 32)
    e2 = _pad_edges(edge_index2, edge_weight2, 32)

    h1 = _matmul(x, W1)
    p1 = _agg_l1(h1, e1)
    h2 = _relu_sum_matmul(p1, W2)
    p2 = _agg_l2(h2, e2)
    return _sum2(p2)


# speedup vs baseline: 1.0582x; 1.0540x over previous
import functools

import jax
import jax.numpy as jnp
from jax import lax
from jax.experimental import pallas as pl
from jax.experimental.pallas import tpu as pltpu
from jax.experimental.pallas import tpu_sc as plsc

N_NODES = 10000
N_EDGES = 320000
NFEAT = 128
NHID = 128
NCLASS = 64

E_PAD1 = 331776
E_PAD2 = 327680
C = 128
ROWS_PER_TILE = N_NODES // 16
_ZCHUNKS = [(0, 128), (128, 128), (256, 128), (384, 128), (512, 113)]


def _mm_body(x_ref, w_ref, o_ref):
    o_ref[...] = jnp.dot(x_ref[...], w_ref[...],
                         preferred_element_type=jnp.float32)


def _matmul(x, w, bm=1000):
    m, k = x.shape
    n = w.shape[1]
    return pl.pallas_call(
        _mm_body,
        grid=(m // bm,),
        in_specs=[pl.BlockSpec((bm, k), lambda i: (i, 0)),
                  pl.BlockSpec((k, n), lambda i: (0, 0))],
        out_specs=pl.BlockSpec((bm, n), lambda i: (i, 0)),
        out_shape=jax.ShapeDtypeStruct((m, n), jnp.float32),
    )(x, w)


def _mm2_body(p_ref, w_ref, o_ref):
    h = jnp.maximum(p_ref[0] + p_ref[1], 0.0)
    o_ref[...] = jnp.dot(h, w_ref[...], preferred_element_type=jnp.float32)


def _relu_sum_matmul(p, w, bm=1000):
    _, m, k = p.shape
    n = w.shape[1]
    return pl.pallas_call(
        _mm2_body,
        grid=(m // bm,),
        in_specs=[pl.BlockSpec((2, bm, k), lambda i: (0, i, 0)),
                  pl.BlockSpec((k, n), lambda i: (0, 0))],
        out_specs=pl.BlockSpec((bm, n), lambda i: (i, 0)),
        out_shape=jax.ShapeDtypeStruct((m, n), jnp.float32),
    )(p, w)


def _sum2_body(p_ref, o_ref):
    o_ref[...] = p_ref[0] + p_ref[1]


def _sum2(p, bm=1000):
    _, m, n = p.shape
    return pl.pallas_call(
        _sum2_body,
        grid=(m // bm,),
        in_specs=[pl.BlockSpec((2, bm, n), lambda i: (0, i, 0))],
        out_specs=pl.BlockSpec((bm, n), lambda i: (i, 0)),
        out_shape=jax.ShapeDtypeStruct((m, n), jnp.float32),
    )(p)


_GATHER_DNUMS = lax.GatherDimensionNumbers(
    offset_dims=(), collapsed_slice_dims=(0,), start_index_map=(0,))


def _lane_splat(vec, lane):
    idx = jnp.full((16, 1), lane, jnp.int32)
    return lax.gather(vec, idx, _GATHER_DNUMS, slice_sizes=(1,),
                      mode=lax.GatherScatterMode.PROMISE_IN_BOUNDS)


def _make_agg(F, R, LA, e_pad):
    nslab = 32
    ept = e_pad // nslab
    nch = ept // C
    assert nch % R == 0 and LA < R
    DW = R - LA
    mesh = plsc.VectorSubcoreMesh(core_axis_name="c", subcore_axis_name="s")

    @functools.partial(
        pl.kernel,
        out_type=jax.ShapeDtypeStruct((2, N_NODES, F), jnp.float32),
        mesh=mesh,
        compiler_params=pltpu.CompilerParams(use_tc_tiling_on_sc=False,
                                             needs_layout_passes=False),
        scratch_types=(
            [pltpu.VMEM_SHARED((N_NODES, F), jnp.float32),
             pltpu.VMEM((R, 3, C), jnp.int32),
             pltpu.VMEM((R, C), jnp.int32)]
            + [pltpu.VMEM((C, F), jnp.float32) for _ in range(R)]
            + [pltpu.SemaphoreType.DMA] * (3 * R)
        ),
    )
    def agg(h_hbm, edges_hbm, out_hbm, acc_sh, stage_v, dstc_v, *rest):
        rows, sems = rest[:R], rest[R:]
        ssems, gsems, scsems = sems[0:R], sems[R:2 * R], sems[2 * R:3 * R]
        c = lax.axis_index("c")
        s = lax.axis_index("s")
        wid = s * 2 + c

        def stage_start(k, b):
            pltpu.async_copy(edges_hbm.at[wid, k], stage_v.at[b], ssems[b])

        def stage_wait(k, b):
            pltpu.make_async_copy(edges_hbm.at[wid, k], stage_v.at[b],
                                  ssems[b]).wait()

        def gather_start(b):
            pltpu.async_copy(h_hbm.at[stage_v.at[b, 0]], rows[b], gsems[b])

        def gather_wait(b):
            pltpu.make_async_copy(h_hbm.at[stage_v.at[b, 0]], rows[b],
                                  gsems[b]).wait()

        def scatter_start(b):
            pltpu.async_copy(rows[b], acc_sh.at[dstc_v.at[b]], scsems[b],
                             add=True)

        def scatter_wait(b):
            pltpu.make_async_copy(rows[b], acc_sh.at[dstc_v.at[b]],
                                  scsems[b]).wait()

        for b in range(R):
            stage_start(b, b)

        def zrow(r, carry):
            for j in range(F // 16):
                rows[0][r, pl.ds(j * 16, 16)] = jnp.zeros((16,), jnp.float32)
            return carry
        lax.fori_loop(0, C, zrow, 0)
        for (z0, zn) in _ZCHUNKS:
            pltpu.sync_copy(rows[0].at[pl.ds(0, zn)],
                            acc_sh.at[pl.ds(s * ROWS_PER_TILE + z0, zn)])
        plsc.subcore_barrier()

        for b in range(LA):
            stage_wait(b, b)
            gather_start(b)

        def step(kk, carry):
            for b in range(R):
                k = kk * R + b
                gather_wait(b)

                def group(g, carry2):
                    wv = plsc.bitcast(stage_v[b, 2, pl.ds(g * 16, 16)],
                                      jnp.float32)
                    for l in range(16):
                        splat = _lane_splat(wv, l)
                        e = g * 16 + l
                        for j in range(F // 16):
                            rows[b][e, pl.ds(j * 16, 16)] = (
                                rows[b][e, pl.ds(j * 16, 16)] * splat)
                    return carry2
                lax.fori_loop(0, C // 16, group, 0)

                for j in range(C // 16):
                    dstc_v[b, pl.ds(j * 16, 16)] = stage_v[b, 1,
                                                           pl.ds(j * 16, 16)]
                scatter_start(b)

                @pl.when(k + R < nch)
                def _():
                    stage_start(k + R, b)

                if b < DW:
                    @pl.when(kk >= 1)
                    def _():
                        scatter_wait((b - DW) % R)
                else:
                    scatter_wait((b - DW) % R)

                @pl.when(k + LA < nch)
                def _():
                    stage_wait(k + LA, (b + LA) % R)
                    gather_start((b + LA) % R)
            return carry
        lax.fori_loop(0, nch // R, step, 0)
        for d in range(DW):
            scatter_wait((nch - DW + d) % R)

        plsc.subcore_barrier()
        for (z0, zn) in _ZCHUNKS:
            r0 = s * ROWS_PER_TILE + z0
            pltpu.sync_copy(acc_sh.at[pl.ds(r0, zn)],
                            out_hbm.at[c, pl.ds(r0, zn)])

    return agg


_agg_l1 = _make_agg(NHID, R=3, LA=2, e_pad=E_PAD1)
_agg_l2 = _make_agg(NCLASS, R=4, LA=2, e_pad=E_PAD2)


def _pad_edges(ei, ew, e_pad):
    nslab = 32
    npad = e_pad - N_EDGES
    nch = e_pad // nslab // C
    spread = jnp.arange(npad, dtype=jnp.int32) % N_NODES
    src = jnp.concatenate([ei[0], spread]).reshape(nslab, nch, 1, C)
    dst = jnp.concatenate([ei[1], spread]).reshape(nslab, nch, 1, C)
    w = lax.bitcast_convert_type(
        jnp.pad(ew, (0, npad)), jnp.int32).reshape(nslab, nch, 1, C)
    return jnp.concatenate([src, dst, w], axis=2)


def kernel(x, edge_index1, edge_index2, edge_weight1, edge_weight2, W1, W2):
    e1 = _pad_edges(edge_index1, edge_weight1, E_PAD1)
    e2 = _pad_edges(edge_index2, edge_weight2, E_PAD2)

    h1 = _matmul(x, W1)
    p1 = _agg_l1(h1, e1)
    h2 = _relu_sum_matmul(p1, W2)
    p2 = _agg_l2(h2, e2)
    return _sum2(p2)
